# baseline (speedup 1.0000x reference)
import functools

import jax
import jax.numpy as jnp
from jax import lax
from jax.experimental import pallas as pl
from jax.experimental.pallas import tpu as pltpu
from jax.experimental.pallas import tpu_sc as plsc

_NC = 2
_NS = 16
_NW = _NC * _NS


def _make_sc_gather(seq, batch, emb, chunk):
    nq = batch // chunk
    nchunk = seq * nq // _NW
    assert nq * chunk == batch and nchunk * _NW == seq * nq
    assert nchunk % 2 == 0
    npair = nchunk // 2
    mesh = plsc.VectorSubcoreMesh(core_axis_name="c", subcore_axis_name="s")

    @functools.partial(
        pl.kernel,
        mesh=mesh,
        out_type=jax.ShapeDtypeStruct((seq, batch, emb), jnp.float32),
        scratch_types=[
            pltpu.VMEM((chunk,), jnp.int32),
            pltpu.VMEM((chunk,), jnp.int32),
            pltpu.VMEM((chunk, emb), jnp.float32),
            pltpu.VMEM((chunk, emb), jnp.float32),
            pltpu.SemaphoreType.DMA,
            pltpu.SemaphoreType.DMA,
        ],
        compiler_params=pltpu.CompilerParams(use_tc_tiling_on_sc=False),
    )
    def k(idx_hbm, table_hbm, out_hbm, idx_v0, idx_v1, rows_v0, rows_v1,
          sem0, sem1):
        wid = lax.axis_index("s") * _NC + lax.axis_index("c")
        base = wid * nchunk

        def stage(c, idx_v, rows_v, sem):
            s, q = c // nq, c % nq
            pltpu.sync_copy(idx_hbm.at[s, pl.ds(q * chunk, chunk)], idx_v)
            pltpu.async_copy(table_hbm.at[idx_v], rows_v, sem)

        def drain(c, idx_v, rows_v, sem):
            s, q = c // nq, c % nq
            pltpu.make_async_copy(table_hbm.at[idx_v], rows_v, sem).wait()
            pltpu.sync_copy(rows_v, out_hbm.at[s, pl.ds(q * chunk, chunk), :])

        stage(base, idx_v0, rows_v0, sem0)

        def body(p, _):
            c0 = base + 2 * p
            stage(c0 + 1, idx_v1, rows_v1, sem1)
            drain(c0, idx_v0, rows_v0, sem0)

            @pl.when(p < npair - 1)
            def _():
                stage(c0 + 2, idx_v0, rows_v0, sem0)

            drain(c0 + 1, idx_v1, rows_v1, sem1)
            return ()

        lax.fori_loop(0, npair, body, ())

    return k


def kernel(input_batch, embeddings):
    seq, batch = input_batch.shape
    vocab, emb = embeddings.shape
    return _make_sc_gather(seq, batch, emb, chunk=512)(
        input_batch.astype(jnp.int32), embeddings)


# speedup vs baseline: 1.6025x; 1.6025x over previous
import functools

import jax
import jax.numpy as jnp
from jax import lax
from jax.experimental import pallas as pl
from jax.experimental.pallas import tpu as pltpu
from jax.experimental.pallas import tpu_sc as plsc

_NC = 2
_NS = 16
_NW = _NC * _NS
_L = 16


def _make_sc_gather_t(seq, batch, emb, vocab):
    assert emb == _NW and batch % _L == 0
    nvec = batch // _L
    mesh = plsc.VectorSubcoreMesh(core_axis_name="c", subcore_axis_name="s")

    @functools.partial(
        pl.kernel,
        mesh=mesh,
        out_type=jax.ShapeDtypeStruct((seq, emb, batch), jnp.float32),
        scratch_types=[
            pltpu.VMEM((vocab,), jnp.float32),
            pltpu.VMEM((batch,), jnp.int32),
            pltpu.VMEM((batch,), jnp.int32),
            pltpu.VMEM((batch,), jnp.float32),
            pltpu.VMEM((batch,), jnp.float32),
            pltpu.SemaphoreType.DMA,
            pltpu.SemaphoreType.DMA,
            pltpu.SemaphoreType.DMA,
            pltpu.SemaphoreType.DMA,
        ],
        compiler_params=pltpu.CompilerParams(
            use_tc_tiling_on_sc=True, needs_layout_passes=False),
    )
    def k(table_t, idx_hbm, out_hbm, row_v, idx_v0, idx_v1, res_v0, res_v1,
          si0, si1, so0, so1):
        e = lax.axis_index("s") * _NC + lax.axis_index("c")
        pltpu.sync_copy(table_t.at[e], row_v)

        def compute(idx_v, res_v):
            for i in range(nvec):
                sl = pl.ds(i * _L, _L)
                res_v[sl] = plsc.load_gather(row_v, [idx_v[sl]])

        pltpu.async_copy(idx_hbm.at[0], idx_v0, si0)

        assert seq % 2 == 0
        npair = seq // 2

        def body(p, _):
            s0 = 2 * p
            s1 = s0 + 1
            pltpu.make_async_copy(idx_hbm.at[s0], idx_v0, si0).wait()
            pltpu.async_copy(idx_hbm.at[s1], idx_v1, si1)

            @pl.when(p > 0)
            def _():
                pltpu.make_async_copy(res_v0, out_hbm.at[s0, e], so0).wait()

            compute(idx_v0, res_v0)
            pltpu.async_copy(res_v0, out_hbm.at[s0, e], so0)

            pltpu.make_async_copy(idx_hbm.at[s1], idx_v1, si1).wait()

            @pl.when(p < npair - 1)
            def _():
                pltpu.async_copy(idx_hbm.at[s0 + 2], idx_v0, si0)

            @pl.when(p > 0)
            def _():
                pltpu.make_async_copy(res_v1, out_hbm.at[s1, e], so1).wait()

            compute(idx_v1, res_v1)
            pltpu.async_copy(res_v1, out_hbm.at[s1, e], so1)
            return ()

        lax.fori_loop(0, npair, body, ())

        pltpu.make_async_copy(res_v0, out_hbm.at[seq - 2, e], so0).wait()
        pltpu.make_async_copy(res_v1, out_hbm.at[seq - 1, e], so1).wait()

    return k


def kernel(input_batch, embeddings):
    seq, batch = input_batch.shape
    vocab, emb = embeddings.shape
    out_t = _make_sc_gather_t(seq, batch, emb, vocab)(
        embeddings.T, input_batch.astype(jnp.int32))
    return out_t.transpose(0, 2, 1)


# speedup vs baseline: 2.2737x; 1.4188x over previous
import functools

import jax
import jax.numpy as jnp
from jax import lax
from jax.experimental import pallas as pl
from jax.experimental.pallas import tpu as pltpu
from jax.experimental.pallas import tpu_sc as plsc

_NC = 2
_NS = 16
_NW = _NC * _NS
_L = 16


def _make_sc_gather_t(seq, batch, emb, vocab):
    assert emb == _NW and batch % _L == 0
    nvec = batch // _L
    mesh = plsc.VectorSubcoreMesh(core_axis_name="c", subcore_axis_name="s")

    @functools.partial(
        pl.kernel,
        mesh=mesh,
        out_type=jax.ShapeDtypeStruct((seq, emb, batch), jnp.float32),
        scratch_types=[
            pltpu.VMEM((vocab,), jnp.float32),
            pltpu.VMEM((batch,), jnp.int32),
            pltpu.VMEM((batch,), jnp.int32),
            pltpu.VMEM((batch,), jnp.float32),
            pltpu.VMEM((batch,), jnp.float32),
            pltpu.SemaphoreType.DMA,
            pltpu.SemaphoreType.DMA,
            pltpu.SemaphoreType.DMA,
            pltpu.SemaphoreType.DMA,
        ],
        compiler_params=pltpu.CompilerParams(
            use_tc_tiling_on_sc=True, needs_layout_passes=False),
    )
    def k(table_t, idx_hbm, out_hbm, row_v, idx_v0, idx_v1, res_v0, res_v1,
          si0, si1, so0, so1):
        e = lax.axis_index("s") * _NC + lax.axis_index("c")
        pltpu.sync_copy(table_t.at[e], row_v)

        def compute(idx_v, res_v):
            @plsc.parallel_loop(0, batch, _L, unroll=8)
            def _(off):
                sl = pl.ds(off, _L)
                res_v[sl] = plsc.load_gather(row_v, [idx_v[sl]])

        pltpu.async_copy(idx_hbm.at[0], idx_v0, si0)

        assert seq % 2 == 0
        npair = seq // 2

        def body(p, _):
            s0 = 2 * p
            s1 = s0 + 1
            pltpu.make_async_copy(idx_hbm.at[s0], idx_v0, si0).wait()
            pltpu.async_copy(idx_hbm.at[s1], idx_v1, si1)

            @pl.when(p > 0)
            def _():
                pltpu.make_async_copy(res_v0, out_hbm.at[s0, e], so0).wait()

            compute(idx_v0, res_v0)
            pltpu.async_copy(res_v0, out_hbm.at[s0, e], so0)

            pltpu.make_async_copy(idx_hbm.at[s1], idx_v1, si1).wait()

            @pl.when(p < npair - 1)
            def _():
                pltpu.async_copy(idx_hbm.at[s0 + 2], idx_v0, si0)

            @pl.when(p > 0)
            def _():
                pltpu.make_async_copy(res_v1, out_hbm.at[s1, e], so1).wait()

            compute(idx_v1, res_v1)
            pltpu.async_copy(res_v1, out_hbm.at[s1, e], so1)
            return ()

        lax.fori_loop(0, npair, body, ())

        pltpu.make_async_copy(res_v0, out_hbm.at[seq - 2, e], so0).wait()
        pltpu.make_async_copy(res_v1, out_hbm.at[seq - 1, e], so1).wait()

    return k


def kernel(input_batch, embeddings):
    seq, batch = input_batch.shape
    vocab, emb = embeddings.shape
    out_t = _make_sc_gather_t(seq, batch, emb, vocab)(
        embeddings.T, input_batch.astype(jnp.int32))
    return out_t.transpose(0, 2, 1)
